# baseline (speedup 1.0000x reference)
import jax
import jax.numpy as jnp
from jax.experimental import pallas as pl

_NODE_TYPES = ["operation", "machine", "job"]
_N = {"operation": 50000, "machine": 1000, "job": 5000}
_EDGE_TYPES = [("operation", "on", "machine"), ("machine", "rev_on", "operation"), ("operation", "belongs", "job"), ("job", "contains", "operation"), ("operation", "precedes", "operation")]
_NUM_LAYERS = 3


def _bn(x, g, b):
    mu = jnp.mean(x, axis=0, keepdims=True)
    var = jnp.var(x, axis=0, keepdims=True)
    return g * (x - mu) * jax.lax.rsqrt(var + 1e-5) + b


def kernel(x_operation, x_machine, x_job, edge_index_operation_on_machine, edge_index_machine_rev_on_operation, edge_index_operation_belongs_job, edge_index_job_contains_operation, edge_index_operation_precedes_operation, valid_pairs, params):
    x_dict = {"operation": x_operation, "machine": x_machine, "job": x_job}
    edge_dict = {
        _EDGE_TYPES[0]: edge_index_operation_on_machine,
        _EDGE_TYPES[1]: edge_index_machine_rev_on_operation,
        _EDGE_TYPES[2]: edge_index_operation_belongs_job,
        _EDGE_TYPES[3]: edge_index_job_contains_operation,
        _EDGE_TYPES[4]: edge_index_operation_precedes_operation,
    }
    x = {}
    for nt in _NODE_TYPES:
        h = (x_dict[nt] @ params["enc_" + nt]) / 1.0
        x[nt] = jnp.concatenate([jnp.sin(h), jnp.cos(h)], axis=-1)
    residuals = []
    for l in range(_NUM_LAYERS):
        outs = {nt: None for nt in _NODE_TYPES}
        for et in _EDGE_TYPES:
            src, rel, dst = et
            ei = edge_dict[et]
            msg = jax.ops.segment_sum(jnp.take(x[src], ei[0], axis=0), ei[1], num_segments=_N[dst])
            h = x[dst] + msg
            p = params["conv%d_%s" % (l, "_".join(et))]
            h = h @ p["W1"] + p["b1"]
            h = _bn(h, p["g1"], p["be1"])
            h = jax.nn.relu(h)
            h = h @ p["W2"] + p["b2"]
            outs[dst] = h if outs[dst] is None else outs[dst] + h
        x_new = {nt: outs[nt] for nt in _NODE_TYPES}
        if residuals:
            x_new = {nt: x_new[nt] + residuals[-1][nt] for nt in _NODE_TYPES}
        residuals.append(x_new)
        x = x_new
    mapping = {"operation": 0, "machine": 1, "job": 2}
    feats = [jnp.take(x[nt], valid_pairs[:, mapping[nt]], axis=0) for nt in _NODE_TYPES]
    cf = jnp.concatenate(feats, axis=1)
    p = params["score"]
    h = _bn(cf @ p["W1"] + p["b1"], p["g1"], p["be1"])
    h = jax.nn.relu(h)
    h = _bn(h @ p["W2"] + p["b2"], p["g2"], p["be2"])
    h = jax.nn.relu(h)
    return h @ p["W3"] + p["b3"]


# speedup vs baseline: 3.8433x; 3.8433x over previous
import functools

import jax
import jax.numpy as jnp
from jax import lax
from jax.experimental import pallas as pl
from jax.experimental.pallas import tpu as pltpu
from jax.experimental.pallas import tpu_sc as plsc

_NODE_TYPES = ["operation", "machine", "job"]
_N = {"operation": 50000, "machine": 1000, "job": 5000}
_NP = {"operation": 50176, "machine": 1024, "job": 5120}
_EDGE_TYPES = [
    ("operation", "on", "machine"),
    ("machine", "rev_on", "operation"),
    ("operation", "belongs", "job"),
    ("job", "contains", "operation"),
    ("operation", "precedes", "operation"),
]
_NUM_LAYERS = 3
_EPS = 1e-5

_NC, _NS = 2, 16
_K = 128
_NSTRIP = 8

_PAIRS = 200000
_PAIRS_P = 200704


def _pad_edges(ei, e_pad):
    e = ei.shape[1]
    row = jnp.concatenate([ei[0], jnp.zeros((e_pad - e,), jnp.int32)])
    col = jnp.concatenate([ei[1], jnp.full((e_pad - e,), 1 << 30, jnp.int32)])
    return row, col


def _e_pad(e):
    blk = _NS * _NSTRIP * 16
    return ((e + blk - 1) // blk) * blk


@functools.cache
def _segsum_kernel(e_pad, n_src, n_dst, f, stage_src):
    ept = e_pad // _NS
    strip = ept // _NSTRIP
    assert strip % 16 == 0 and strip % 8 == 0
    t = strip // _K + 1
    h = n_dst // 2
    q, rem = divmod(h, _NS)
    assert q % 8 == 0 and rem % 8 == 0, (q, rem)
    qs, rems = divmod(n_src, _NS)
    mesh = plsc.VectorSubcoreMesh(
        core_axis_name="c", subcore_axis_name="s",
        num_cores=_NC, num_subcores=_NS)

    scratch = [
        pltpu.VMEM_SHARED((h + 16, f), jnp.float32),
        pltpu.VMEM((strip,), jnp.int32),
        pltpu.VMEM((strip,), jnp.int32),
        pltpu.VMEM((t, _K), jnp.int32),
        pltpu.VMEM((t, _K), jnp.int32),
        pltpu.VMEM((_K, f), jnp.float32),
        pltpu.SemaphoreType.DMA,
    ]
    if stage_src:
        scratch.insert(1, pltpu.VMEM_SHARED((n_src, f), jnp.float32))

    @functools.partial(
        pl.kernel,
        out_type=jax.ShapeDtypeStruct((n_dst, f), jnp.float32),
        mesh=mesh,
        scratch_types=scratch,
        compiler_params=pltpu.CompilerParams(
            needs_layout_passes=False, use_tc_tiling_on_sc=False),
    )
    def k(row_hbm, col_hbm, xdst_hbm, src_hbm, out_hbm, acc, *rest):
        if stage_src:
            stage, rowb, colb, crow, ccol, gbuf, gsem = rest
        else:
            rowb, colb, crow, ccol, gbuf, gsem = rest
            stage = None
        c = lax.axis_index("c")
        s = lax.axis_index("s")
        lo = c * h

        pltpu.sync_copy(xdst_hbm.at[pl.ds(lo + s * q, q)],
                        acc.at[pl.ds(s * q, q)])
        if rem:
            @pl.when(s == 0)
            def _():
                pltpu.sync_copy(xdst_hbm.at[pl.ds(lo + _NS * q, rem)],
                                acc.at[pl.ds(_NS * q, rem)])
        if stage_src:
            pltpu.sync_copy(src_hbm.at[pl.ds(s * qs, qs)],
                            stage.at[pl.ds(s * qs, qs)])
            if rems:
                @pl.when(s == 0)
                def _():
                    pltpu.sync_copy(src_hbm.at[pl.ds(_NS * qs, rems)],
                                    stage.at[pl.ds(_NS * qs, rems)])
        plsc.subcore_barrier()

        lo_v = jnp.full((16,), lo, jnp.int32)
        hi_v = jnp.full((16,), lo + h, jnp.int32)
        iota = lax.iota(jnp.int32, 16)
        gsrc = stage if stage_src else src_hbm

        def strip_body(st, _):
            base_e = s * ept + st * strip
            pltpu.sync_copy(row_hbm.at[pl.ds(base_e, strip)], rowb)
            pltpu.sync_copy(col_hbm.at[pl.ds(base_e, strip)], colb)

            def scan_body(v, cnt):
                colv = colb[pl.ds(v * 16, 16)]
                rowv = rowb[pl.ds(v * 16, 16)]
                m = (colv >= lo_v) & (colv < hi_v)
                pc = plsc.cumsum(m.astype(jnp.int32))
                idx = cnt + pc - 1
                plsc.store_scatter(crow, [idx >> 7, idx & 127], rowv, mask=m)
                plsc.store_scatter(ccol, [idx >> 7, idx & 127], colv - lo_v,
                                   mask=m)
                return cnt + plsc.all_reduce_population_count(m)

            cnt = lax.fori_loop(0, strip // 16, scan_body,
                                jnp.zeros((16,), jnp.int32))

            cnts = jnp.max(cnt)
            trip = (cnts + (_K - 1)) // _K
            total_v = jnp.full((16,), trip * _K, jnp.int32)

            def pad_body(j, _):
                idxv = cnt + j * 16 + iota
                m = idxv < total_v
                plsc.store_scatter(crow, [idxv >> 7, idxv & 127], iota,
                                   mask=m)
                plsc.store_scatter(ccol, [idxv >> 7, idxv & 127],
                                   jnp.full((16,), h, jnp.int32) + iota,
                                   mask=m)
                return 0

            lax.fori_loop(0, _K // 16, pad_body, 0)

            def gs_body(j, _):
                pltpu.async_copy(gsrc.at[crow.at[j]], gbuf, gsem).wait()
                pltpu.sync_copy(gbuf, acc.at[ccol.at[j]], add=True)
                return 0

            lax.fori_loop(0, trip, gs_body, 0)
            return 0

        lax.fori_loop(0, _NSTRIP, strip_body, 0)
        plsc.subcore_barrier()

        pltpu.sync_copy(acc.at[pl.ds(s * q, q)],
                        out_hbm.at[pl.ds(lo + s * q, q)])
        if rem:
            @pl.when(s == 0)
            def _():
                pltpu.sync_copy(acc.at[pl.ds(_NS * q, rem)],
                                out_hbm.at[pl.ds(lo + _NS * q, rem)])

    return k


def _segsum(row, col, x_src, x_dst, stage_src):
    k = _segsum_kernel(row.shape[0], x_src.shape[0], x_dst.shape[0],
                       x_src.shape[1], stage_src)
    return k(row, col, x_dst, x_src)


@functools.cache
def _pair_gather_kernel():
    per_w = _PAIRS_P // (_NC * _NS)
    ntrip = per_w // _K
    f = 64
    shapes = [_NP["operation"], _NP["machine"], _NP["job"]]
    out = [jax.ShapeDtypeStruct((_PAIRS_P, f), jnp.float32) for _ in range(3)]
    scratch = [
        pltpu.VMEM_SHARED((shapes[1], f), jnp.float32),
        pltpu.VMEM_SHARED((shapes[2], f), jnp.float32),
        pltpu.VMEM((per_w,), jnp.int32),
        pltpu.VMEM((per_w,), jnp.int32),
        pltpu.VMEM((per_w,), jnp.int32),
        pltpu.VMEM((_K, f), jnp.float32),
        pltpu.VMEM((_K, f), jnp.float32),
        pltpu.VMEM((_K, f), jnp.float32),
        pltpu.SemaphoreType.DMA,
        pltpu.SemaphoreType.DMA,
        pltpu.SemaphoreType.DMA,
    ]
    mesh = plsc.VectorSubcoreMesh(
        core_axis_name="c", subcore_axis_name="s",
        num_cores=_NC, num_subcores=_NS)

    @functools.partial(
        pl.kernel, out_type=out, mesh=mesh, scratch_types=scratch,
        compiler_params=pltpu.CompilerParams(
            needs_layout_passes=False, use_tc_tiling_on_sc=False),
    )
    def k(t_op, t_ma, t_job, i0, i1, i2, g0, g1, g2,
          st_ma, st_job, b0, b1, b2, v0, v1, v2, s0, s1, s2):
        c = lax.axis_index("c")
        s = lax.axis_index("s")
        wid = s * _NC + c
        qm = shapes[1] // _NS
        qj = shapes[2] // _NS
        pltpu.sync_copy(t_ma.at[pl.ds(s * qm, qm)], st_ma.at[pl.ds(s * qm, qm)])
        pltpu.sync_copy(t_job.at[pl.ds(s * qj, qj)],
                        st_job.at[pl.ds(s * qj, qj)])
        plsc.subcore_barrier()

        base = wid * per_w
        pltpu.sync_copy(i0.at[pl.ds(base, per_w)], b0)
        pltpu.sync_copy(i1.at[pl.ds(base, per_w)], b1)
        pltpu.sync_copy(i2.at[pl.ds(base, per_w)], b2)

        def chunk(j, _):
            o = j * _K
            d0 = pltpu.async_copy(t_op.at[b0.at[pl.ds(o, _K)]], v0, s0)
            d1 = pltpu.async_copy(st_ma.at[b1.at[pl.ds(o, _K)]], v1, s1)
            d2 = pltpu.async_copy(st_job.at[b2.at[pl.ds(o, _K)]], v2, s2)
            d0.wait()
            d1.wait()
            d2.wait()
            pltpu.sync_copy(v0, g0.at[pl.ds(base + o, _K)])
            pltpu.sync_copy(v1, g1.at[pl.ds(base + o, _K)])
            pltpu.sync_copy(v2, g2.at[pl.ds(base + o, _K)])
            return 0

        lax.fori_loop(0, ntrip, chunk, 0)

    return k


def _row_mask(vals, base, n_valid):
    rid = base + lax.broadcasted_iota(jnp.int32, vals.shape, 0)
    return jnp.where(rid < n_valid, vals, 0.0)


@functools.cache
def _encoder_kernel(n_p, n, inch, bs):
    grid = n_p // bs

    def body(x_ref, w_ref, o_ref):
        i = pl.program_id(0)
        hx = jnp.dot(x_ref[...], w_ref[...],
                     preferred_element_type=jnp.float32)
        enc = jnp.concatenate([jnp.sin(hx), jnp.cos(hx)], axis=1)
        o_ref[...] = _row_mask(enc, i * bs, n)

    return pl.pallas_call(
        body,
        grid=(grid,),
        in_specs=[pl.BlockSpec((bs, inch), lambda i: (i, 0)),
                  pl.BlockSpec((inch, 16), lambda i: (0, 0))],
        out_specs=pl.BlockSpec((bs, 32), lambda i: (i, 0)),
        out_shape=jax.ShapeDtypeStruct((n_p, 32), jnp.float32),
    )


@functools.cache
def _conv3a_kernel(n_p, n, f_in, bs):
    grid = n_p // bs
    f = 64

    def body(z0_ref, z1_ref, z2_ref, w1_ref, b1_ref,
             h0_ref, h1_ref, h2_ref, s1_ref, s2_ref, ctr_ref):
        i = pl.program_id(0)
        ones = jnp.ones((1, bs), jnp.float32)
        hrefs = (h0_ref, h1_ref, h2_ref)

        @pl.when(i == 0)
        def _():
            s1_ref[...] = jnp.zeros_like(s1_ref)
            s2_ref[...] = jnp.zeros_like(s2_ref)

        for e, zr in enumerate((z0_ref, z1_ref, z2_ref)):
            he = jnp.dot(zr[...], w1_ref[e],
                         preferred_element_type=jnp.float32) + b1_ref[e, 0:1, :]
            he = _row_mask(he, i * bs, n)
            hrefs[e][...] = he

            @pl.when(i == 0)
            def _():
                cm = jnp.dot(ones, he, preferred_element_type=jnp.float32,
                             precision=lax.Precision.HIGHEST) * (1.0 / bs)
                ctr_ref[e] = jnp.broadcast_to(cm, (8, f))

            hc = _row_mask(he - ctr_ref[e, 0:1, :], i * bs, n)
            s1_ref[e, 0:1, :] += jnp.dot(
                ones, he, preferred_element_type=jnp.float32,
                precision=lax.Precision.HIGHEST)
            s2_ref[e, 0:1, :] += jnp.dot(
                ones, hc * hc, preferred_element_type=jnp.float32,
                precision=lax.Precision.HIGHEST)

    return pl.pallas_call(
        body,
        grid=(grid,),
        in_specs=[pl.BlockSpec((bs, f_in), lambda i: (i, 0))] * 3
        + [pl.BlockSpec((3, f_in, f), lambda i: (0, 0, 0)),
           pl.BlockSpec((3, 8, f), lambda i: (0, 0, 0))],
        out_specs=[pl.BlockSpec((bs, f), lambda i: (i, 0))] * 3
        + [pl.BlockSpec((3, 8, f), lambda i: (0, 0, 0)),
           pl.BlockSpec((3, 8, f), lambda i: (0, 0, 0)),
           pl.BlockSpec((3, 8, f), lambda i: (0, 0, 0))],
        out_shape=[jax.ShapeDtypeStruct((n_p, f), jnp.float32)] * 3
        + [jax.ShapeDtypeStruct((3, 8, f), jnp.float32)] * 3,
    )


@functools.cache
def _conv3b_kernel(n_p, n, bs, has_res):
    grid = n_p // bs
    f = 64

    def body(*refs):
        if has_res:
            h0_ref, h1_ref, h2_ref, res_ref, so_ref, w2_ref, b2_ref, o_ref = refs
        else:
            h0_ref, h1_ref, h2_ref, so_ref, w2_ref, b2_ref, o_ref = refs
            res_ref = None
        i = pl.program_id(0)
        acc = jnp.broadcast_to(b2_ref[0:1, :], (bs, f))
        if has_res:
            acc = acc + res_ref[...]
        for e, hr in enumerate((h0_ref, h1_ref, h2_ref)):
            hb = jnp.maximum(hr[...] * so_ref[e, 0:1, :] + so_ref[e, 1:2, :],
                             0.0)
            acc = acc + jnp.dot(hb, w2_ref[e],
                                preferred_element_type=jnp.float32)
        o_ref[...] = _row_mask(acc, i * bs, n)

    in_specs = [pl.BlockSpec((bs, f), lambda i: (i, 0))] * 3
    if has_res:
        in_specs.append(pl.BlockSpec((bs, f), lambda i: (i, 0)))
    in_specs += [
        pl.BlockSpec((3, 8, f), lambda i: (0, 0, 0)),
        pl.BlockSpec((3, f, f), lambda i: (0, 0, 0)),
        pl.BlockSpec((8, f), lambda i: (0, 0)),
    ]
    return pl.pallas_call(
        body,
        grid=(grid,),
        in_specs=in_specs,
        out_specs=pl.BlockSpec((bs, f), lambda i: (i, 0)),
        out_shape=jax.ShapeDtypeStruct((n_p, f), jnp.float32),
    )


@functools.cache
def _conv1_kernel(n_p, n, f_in, has_res):
    def body(*refs):
        if has_res:
            z_ref, res_ref, w1_ref, w2_ref, vec_ref, o_ref = refs
        else:
            z_ref, w1_ref, w2_ref, vec_ref, o_ref = refs
            res_ref = None
        b1 = vec_ref[0:1, :]
        g1 = vec_ref[1:2, :]
        be1 = vec_ref[2:3, :]
        b2 = vec_ref[3:4, :]
        z = z_ref[...]
        h1 = jnp.dot(z, w1_ref[...], preferred_element_type=jnp.float32) + b1
        npad = n_p - n
        mu = (h1.sum(0, keepdims=True) - npad * b1) * (1.0 / n)
        d = h1 - mu
        var = ((d * d).sum(0, keepdims=True)
               - npad * (b1 - mu) * (b1 - mu)) * (1.0 / n)
        hb = jnp.maximum(g1 * d * lax.rsqrt(var + _EPS) + be1, 0.0)
        out = jnp.dot(hb, w2_ref[...], preferred_element_type=jnp.float32) + b2
        if has_res:
            out = out + res_ref[...]
        o_ref[...] = _row_mask(out, 0, n)

    in_specs = [pl.BlockSpec((n_p, f_in), lambda: (0, 0))]
    if has_res:
        in_specs.append(pl.BlockSpec((n_p, 64), lambda: (0, 0)))
    in_specs += [
        pl.BlockSpec((f_in, 64), lambda: (0, 0)),
        pl.BlockSpec((64, 64), lambda: (0, 0)),
        pl.BlockSpec((8, 64), lambda: (0, 0)),
    ]
    return pl.pallas_call(
        body,
        grid=(),
        in_specs=in_specs,
        out_specs=pl.BlockSpec((n_p, 64), lambda: (0, 0)),
        out_shape=jax.ShapeDtypeStruct((n_p, 64), jnp.float32),
    )


@functools.cache
def _linear_kernel(n_p, f_in, f_out, bs):
    grid = n_p // bs

    def body(x_ref, w_ref, o_ref):
        o_ref[...] = jnp.dot(x_ref[...], w_ref[...],
                             preferred_element_type=jnp.float32)

    return pl.pallas_call(
        body,
        grid=(grid,),
        in_specs=[pl.BlockSpec((bs, f_in), lambda i: (i, 0)),
                  pl.BlockSpec((f_in, f_out), lambda i: (0, 0))],
        out_specs=pl.BlockSpec((bs, f_out), lambda i: (i, 0)),
        out_shape=jax.ShapeDtypeStruct((n_p, f_out), jnp.float32),
    )


@functools.cache
def _score_stats_kernel(bs):
    grid = _PAIRS_P // bs
    f = 64

    def body(g0_ref, g1_ref, g2_ref, b1_ref, h_ref, s1_ref, s2_ref,
             ctr_ref):
        i = pl.program_id(0)
        ones = jnp.ones((1, bs), jnp.float32)

        h = g0_ref[...] + g1_ref[...] + g2_ref[...] + b1_ref[0:1, :]
        h = _row_mask(h, i * bs, _PAIRS)
        h_ref[...] = h

        @pl.when(i == 0)
        def _():
            s1_ref[...] = jnp.zeros_like(s1_ref)
            s2_ref[...] = jnp.zeros_like(s2_ref)
            cm = jnp.dot(ones, h, preferred_element_type=jnp.float32,
                         precision=lax.Precision.HIGHEST) * (1.0 / bs)
            ctr_ref[...] = jnp.broadcast_to(cm, (8, f))

        hc = _row_mask(h - ctr_ref[0:1, :], i * bs, _PAIRS)
        s1_ref[0:1, :] += jnp.dot(ones, h, preferred_element_type=jnp.float32,
                                  precision=lax.Precision.HIGHEST)
        s2_ref[0:1, :] += jnp.dot(ones, hc * hc,
                                  preferred_element_type=jnp.float32,
                                  precision=lax.Precision.HIGHEST)

    return pl.pallas_call(
        body,
        grid=(grid,),
        in_specs=[pl.BlockSpec((bs, f), lambda i: (i, 0))] * 3
        + [pl.BlockSpec((8, f), lambda i: (0, 0))],
        out_specs=[pl.BlockSpec((bs, f), lambda i: (i, 0)),
                   pl.BlockSpec((8, f), lambda i: (0, 0)),
                   pl.BlockSpec((8, f), lambda i: (0, 0)),
                   pl.BlockSpec((8, f), lambda i: (0, 0))],
        out_shape=[jax.ShapeDtypeStruct((_PAIRS_P, f), jnp.float32),
                   jax.ShapeDtypeStruct((8, f), jnp.float32),
                   jax.ShapeDtypeStruct((8, f), jnp.float32),
                   jax.ShapeDtypeStruct((8, f), jnp.float32)],
    )


@functools.cache
def _score_pass2_kernel(bs):
    grid = _PAIRS_P // bs
    f, f2 = 64, 32

    def body(h_ref, sc_ref, of_ref, w2_ref, b2_ref, h2_ref, s1_ref, s2_ref,
             ctr_ref):
        i = pl.program_id(0)
        ones = jnp.ones((1, bs), jnp.float32)

        hb = jnp.maximum(h_ref[...] * sc_ref[0:1, :] + of_ref[0:1, :], 0.0)
        h2 = jnp.dot(hb, w2_ref[...],
                     preferred_element_type=jnp.float32) + b2_ref[0:1, :]
        h2 = _row_mask(h2, i * bs, _PAIRS)
        h2_ref[...] = h2

        @pl.when(i == 0)
        def _():
            s1_ref[...] = jnp.zeros_like(s1_ref)
            s2_ref[...] = jnp.zeros_like(s2_ref)
            cm = jnp.dot(ones, h2, preferred_element_type=jnp.float32,
                         precision=lax.Precision.HIGHEST) * (1.0 / bs)
            ctr_ref[...] = jnp.broadcast_to(cm, (8, f2))

        hc = _row_mask(h2 - ctr_ref[0:1, :], i * bs, _PAIRS)
        s1_ref[0:1, :] += jnp.dot(ones, h2, preferred_element_type=jnp.float32,
                                  precision=lax.Precision.HIGHEST)
        s2_ref[0:1, :] += jnp.dot(ones, hc * hc,
                                  preferred_element_type=jnp.float32,
                                  precision=lax.Precision.HIGHEST)

    return pl.pallas_call(
        body,
        grid=(grid,),
        in_specs=[pl.BlockSpec((bs, f), lambda i: (i, 0)),
                  pl.BlockSpec((8, f), lambda i: (0, 0)),
                  pl.BlockSpec((8, f), lambda i: (0, 0)),
                  pl.BlockSpec((f, f2), lambda i: (0, 0)),
                  pl.BlockSpec((8, f2), lambda i: (0, 0))],
        out_specs=[pl.BlockSpec((bs, f2), lambda i: (i, 0)),
                   pl.BlockSpec((8, f2), lambda i: (0, 0)),
                   pl.BlockSpec((8, f2), lambda i: (0, 0)),
                   pl.BlockSpec((8, f2), lambda i: (0, 0))],
        out_shape=[jax.ShapeDtypeStruct((_PAIRS_P, f2), jnp.float32),
                   jax.ShapeDtypeStruct((8, f2), jnp.float32),
                   jax.ShapeDtypeStruct((8, f2), jnp.float32),
                   jax.ShapeDtypeStruct((8, f2), jnp.float32)],
    )


@functools.cache
def _score_pass3_kernel(bs):
    grid = _PAIRS_P // bs
    f2 = 32

    def body(h2_ref, sc_ref, of_ref, w3_ref, b3_ref, o_ref):
        hb = jnp.maximum(h2_ref[...] * sc_ref[0:1, :] + of_ref[0:1, :], 0.0)
        o_ref[...] = jnp.dot(hb, w3_ref[...],
                             preferred_element_type=jnp.float32) + b3_ref[0:1, :]

    return pl.pallas_call(
        body,
        grid=(grid,),
        in_specs=[pl.BlockSpec((bs, f2), lambda i: (i, 0)),
                  pl.BlockSpec((8, f2), lambda i: (0, 0)),
                  pl.BlockSpec((8, f2), lambda i: (0, 0)),
                  pl.BlockSpec((f2, 1), lambda i: (0, 0)),
                  pl.BlockSpec((8, 1), lambda i: (0, 0))],
        out_specs=pl.BlockSpec((bs, 1), lambda i: (i, 0)),
        out_shape=jax.ShapeDtypeStruct((_PAIRS_P, 1), jnp.float32),
    )


def _vec8(*rows):
    f = rows[0].shape[-1]
    v = jnp.zeros((8, f), jnp.float32)
    for r, x in enumerate(rows):
        v = v.at[r].set(x.reshape(f))
    return v


def kernel(x_operation, x_machine, x_job, edge_index_operation_on_machine, edge_index_machine_rev_on_operation, edge_index_operation_belongs_job, edge_index_job_contains_operation, edge_index_operation_precedes_operation, valid_pairs, params):
    edge_dict = {
        _EDGE_TYPES[0]: edge_index_operation_on_machine,
        _EDGE_TYPES[1]: edge_index_machine_rev_on_operation,
        _EDGE_TYPES[2]: edge_index_operation_belongs_job,
        _EDGE_TYPES[3]: edge_index_job_contains_operation,
        _EDGE_TYPES[4]: edge_index_operation_precedes_operation,
    }
    raw = {"operation": x_operation, "machine": x_machine, "job": x_job}

    epad = {}
    for et in _EDGE_TYPES:
        ei = edge_dict[et]
        epad[et] = _pad_edges(ei, _e_pad(ei.shape[1]))

    x = {}
    for nt in _NODE_TYPES:
        xr = raw[nt]
        n, n_p = _N[nt], _NP[nt]
        xr_p = jnp.pad(xr, ((0, n_p - n), (0, 8 - xr.shape[1])))
        enc_p = jnp.pad(params["enc_" + nt], ((0, 8 - xr.shape[1]), (0, 0)))
        bs = {"operation": 1568, "machine": 1024, "job": 1024}[nt]
        x[nt] = _encoder_kernel(n_p, n, 8, bs)(xr_p, enc_p)

    op_ets = [_EDGE_TYPES[1], _EDGE_TYPES[3], _EDGE_TYPES[4]]
    prev = None
    for l in range(_NUM_LAYERS):
        f = 32 if l == 0 else 64
        z = {}
        for et in _EDGE_TYPES:
            src, _, dst = et
            row, col = epad[et]
            z[et] = _segsum(row, col, x[src], x[dst],
                            stage_src=(src == "machine"))

        x_new = {}
        zs = [z[et] for et in op_ets]
        pre = [params["conv%d_%s" % (l, "_".join(et))] for et in op_ets]
        w1s = jnp.stack([p["W1"] for p in pre])
        w2s = jnp.stack([p["W2"] for p in pre])
        b1s = jnp.stack([_vec8(p["b1"]) for p in pre])
        np_op, n_op = _NP["operation"], _N["operation"]
        h0, h1_, h2_, s1c, s2c, ctrc = _conv3a_kernel(np_op, n_op, f, 1568)(
            *zs, w1s, b1s)
        so = []
        for e, p in enumerate(pre):
            mu = s1c[e, 0] / n_op
            dd = mu - ctrc[e, 0]
            var = s2c[e, 0] / n_op - dd * dd
            scale = p["g1"] * lax.rsqrt(var + _EPS)
            so.append(jnp.stack([scale, p["be1"] - mu * scale]))
        sov = jnp.stack([jnp.pad(s_, ((0, 6), (0, 0))) for s_ in so])
        b2sum = _vec8(pre[0]["b2"] + pre[1]["b2"] + pre[2]["b2"])
        args = [h0, h1_, h2_] + ([prev["operation"]] if prev else []) + [
            sov, w2s, b2sum]
        x_new["operation"] = _conv3b_kernel(np_op, n_op, 1568,
                                            prev is not None)(*args)

        for nt, et in (("machine", _EDGE_TYPES[0]), ("job", _EDGE_TYPES[2])):
            p = params["conv%d_%s" % (l, "_".join(et))]
            vec = _vec8(p["b1"], p["g1"], p["be1"], p["b2"])
            args = [z[et]] + ([prev[nt]] if prev else []) + [p["W1"], p["W2"],
                                                             vec]
            x_new[nt] = _conv1_kernel(_NP[nt], _N[nt], f,
                                      prev is not None)(*args)
        prev = x_new
        x = x_new

    sp = params["score"]
    p_op = _linear_kernel(_NP["operation"], 64, 64, 1568)(
        x["operation"], sp["W1"][0:64])
    p_ma = _linear_kernel(_NP["machine"], 64, 64, 1024)(
        x["machine"], sp["W1"][64:128])
    p_job = _linear_kernel(_NP["job"], 64, 64, 1024)(
        x["job"], sp["W1"][128:192])

    idx = [jnp.pad(valid_pairs[:, j], (0, _PAIRS_P - _PAIRS)) for j in range(3)]
    g0, g1, g2 = _pair_gather_kernel()(p_op, p_ma, p_job, *idx)

    b1v = _vec8(sp["b1"])
    h1, s1, s2, c1 = _score_stats_kernel(1568)(g0, g1, g2, b1v)
    mu1 = s1[0] / _PAIRS
    d1 = mu1 - c1[0]
    var1 = s2[0] / _PAIRS - d1 * d1
    sc1 = sp["g1"] * lax.rsqrt(var1 + _EPS)
    of1 = sp["be1"] - mu1 * sc1

    h2, t1, t2, c2 = _score_pass2_kernel(1568)(
        h1, _vec8(sc1), _vec8(of1), sp["W2"], _vec8(sp["b2"]))
    mu2 = t1[0] / _PAIRS
    d2 = mu2 - c2[0]
    var2 = t2[0] / _PAIRS - d2 * d2
    sc2 = sp["g2"] * lax.rsqrt(var2 + _EPS)
    of2 = sp["be2"] - mu2 * sc2

    out = _score_pass3_kernel(1568)(
        h2, _vec8(sc2), _vec8(of2), sp["W3"],
        jnp.broadcast_to(sp["b3"].reshape(1, 1), (8, 1)))
    return out[:_PAIRS]


# speedup vs baseline: 4.3705x; 1.1372x over previous
import functools

import jax
import jax.numpy as jnp
from jax import lax
from jax.experimental import pallas as pl
from jax.experimental.pallas import tpu as pltpu
from jax.experimental.pallas import tpu_sc as plsc

_NODE_TYPES = ["operation", "machine", "job"]
_N = {"operation": 50000, "machine": 1000, "job": 5000}
_NP = {"operation": 50176, "machine": 1024, "job": 5120}
_EDGE_TYPES = [
    ("operation", "on", "machine"),
    ("machine", "rev_on", "operation"),
    ("operation", "belongs", "job"),
    ("job", "contains", "operation"),
    ("operation", "precedes", "operation"),
]
_NUM_LAYERS = 3
_EPS = 1e-5

_NC, _NS = 2, 16
_K = 128
_NSTRIP = 8

_PAIRS = 200000
_PAIRS_P = 200704


def _pad_edges(ei, e_pad):
    e = ei.shape[1]
    row = jnp.concatenate([ei[0], jnp.zeros((e_pad - e,), jnp.int32)])
    col = jnp.concatenate([ei[1], jnp.full((e_pad - e,), 1 << 30, jnp.int32)])
    return row, col


def _e_pad(e):
    blk = _NS * _NSTRIP * 16
    return ((e + blk - 1) // blk) * blk


@functools.cache
def _segsum_kernel(e_pad, n_src, n_dst, f, stage_src):
    ept = e_pad // _NS
    nstrip = max(1, min(_NSTRIP, ept // 3136))
    while ept % nstrip or (ept // nstrip) % 16:
        nstrip -= 1
    strip = ept // nstrip
    assert strip % 16 == 0 and strip % 8 == 0
    t = strip // _K + 1
    h = n_dst // 2
    q, rem = divmod(h, _NS)
    assert q % 8 == 0 and rem % 8 == 0, (q, rem)
    qs, rems = divmod(n_src, _NS)
    mesh = plsc.VectorSubcoreMesh(
        core_axis_name="c", subcore_axis_name="s",
        num_cores=_NC, num_subcores=_NS)

    scratch = [
        pltpu.VMEM_SHARED((h + 16, f), jnp.float32),
        pltpu.VMEM((strip,), jnp.int32),
        pltpu.VMEM((strip,), jnp.int32),
        pltpu.VMEM((t, _K), jnp.int32),
        pltpu.VMEM((t, _K), jnp.int32),
        pltpu.VMEM((_K, f), jnp.float32),
        pltpu.SemaphoreType.DMA,
        pltpu.SemaphoreType.DMA,
    ]
    if not stage_src:
        scratch.insert(-2, pltpu.VMEM((_K, f), jnp.float32))
    if stage_src:
        scratch.insert(1, pltpu.VMEM_SHARED((n_src, f), jnp.float32))

    @functools.partial(
        pl.kernel,
        out_type=jax.ShapeDtypeStruct((n_dst, f), jnp.float32),
        mesh=mesh,
        scratch_types=scratch,
        compiler_params=pltpu.CompilerParams(
            needs_layout_passes=False, use_tc_tiling_on_sc=False),
    )
    def k(row_hbm, col_hbm, xdst_hbm, src_hbm, out_hbm, acc, *rest):
        if stage_src:
            stage, rowb, colb, crow, ccol, gb0, gs0, gs1 = rest
            gb1 = None
        else:
            rowb, colb, crow, ccol, gb0, gb1, gs0, gs1 = rest
            stage = None
        c = lax.axis_index("c")
        s = lax.axis_index("s")
        lo = c * h

        pltpu.sync_copy(xdst_hbm.at[pl.ds(lo + s * q, q)],
                        acc.at[pl.ds(s * q, q)])
        if rem:
            @pl.when(s == 0)
            def _():
                pltpu.sync_copy(xdst_hbm.at[pl.ds(lo + _NS * q, rem)],
                                acc.at[pl.ds(_NS * q, rem)])
        if stage_src:
            pltpu.sync_copy(src_hbm.at[pl.ds(s * qs, qs)],
                            stage.at[pl.ds(s * qs, qs)])
            if rems:
                @pl.when(s == 0)
                def _():
                    pltpu.sync_copy(src_hbm.at[pl.ds(_NS * qs, rems)],
                                    stage.at[pl.ds(_NS * qs, rems)])
        plsc.subcore_barrier()

        lo_v = jnp.full((16,), lo, jnp.int32)
        hi_v = jnp.full((16,), lo + h, jnp.int32)
        iota = lax.iota(jnp.int32, 16)
        gsrc = stage if stage_src else src_hbm

        def strip_body(st, _):
            base_e = s * ept + st * strip
            dr = pltpu.async_copy(row_hbm.at[pl.ds(base_e, strip)], rowb, gs0)
            dc = pltpu.async_copy(col_hbm.at[pl.ds(base_e, strip)], colb, gs1)
            dr.wait()
            dc.wait()

            def scan_body(v, cnt):
                colv = colb[pl.ds(v * 16, 16)]
                rowv = rowb[pl.ds(v * 16, 16)]
                m = (colv >= lo_v) & (colv < hi_v)
                pc = plsc.cumsum(m.astype(jnp.int32))
                idx = cnt + pc - 1
                plsc.store_scatter(crow, [idx >> 7, idx & 127], rowv, mask=m)
                plsc.store_scatter(ccol, [idx >> 7, idx & 127], colv - lo_v,
                                   mask=m)
                return cnt + plsc.all_reduce_population_count(m)

            cnt = lax.fori_loop(0, strip // 16, scan_body,
                                jnp.zeros((16,), jnp.int32))

            cnts = jnp.max(cnt)
            trip = (cnts + (_K - 1)) // _K
            total_v = jnp.full((16,), trip * _K, jnp.int32)

            def pad_body(j, _):
                idxv = cnt + j * 16 + iota
                m = idxv < total_v
                plsc.store_scatter(crow, [idxv >> 7, idxv & 127], iota,
                                   mask=m)
                plsc.store_scatter(ccol, [idxv >> 7, idxv & 127],
                                   jnp.full((16,), h, jnp.int32) + iota,
                                   mask=m)
                return 0

            lax.fori_loop(0, _K // 16, pad_body, 0)

            if stage_src:
                def gs_seq(j, _):
                    pltpu.async_copy(gsrc.at[crow.at[j]], gb0, gs0).wait()
                    pltpu.sync_copy(gb0, acc.at[ccol.at[j]], add=True)
                    return 0

                lax.fori_loop(0, trip, gs_seq, 0)
                return 0

            @pl.when(trip > 0)
            def _():
                pltpu.async_copy(gsrc.at[crow.at[0]], gb0, gs0)

            def gs_body(j, _):
                even = lax.rem(j, 2) == 0

                @pl.when(even)
                def _():
                    pltpu.make_async_copy(gsrc.at[crow.at[j]], gb0, gs0).wait()

                    @pl.when(j + 1 < trip)
                    def _():
                        pltpu.async_copy(gsrc.at[crow.at[j + 1]], gb1, gs1)
                    pltpu.sync_copy(gb0, acc.at[ccol.at[j]], add=True)

                @pl.when(jnp.logical_not(even))
                def _():
                    pltpu.make_async_copy(gsrc.at[crow.at[j]], gb1, gs1).wait()

                    @pl.when(j + 1 < trip)
                    def _():
                        pltpu.async_copy(gsrc.at[crow.at[j + 1]], gb0, gs0)
                    pltpu.sync_copy(gb1, acc.at[ccol.at[j]], add=True)
                return 0

            lax.fori_loop(0, trip, gs_body, 0)
            return 0

        lax.fori_loop(0, nstrip, strip_body, 0)
        plsc.subcore_barrier()

        pltpu.sync_copy(acc.at[pl.ds(s * q, q)],
                        out_hbm.at[pl.ds(lo + s * q, q)])
        if rem:
            @pl.when(s == 0)
            def _():
                pltpu.sync_copy(acc.at[pl.ds(_NS * q, rem)],
                                out_hbm.at[pl.ds(lo + _NS * q, rem)])

    return k


def _segsum(row, col, x_src, x_dst, stage_src):
    k = _segsum_kernel(row.shape[0], x_src.shape[0], x_dst.shape[0],
                       x_src.shape[1], stage_src)
    return k(row, col, x_dst, x_src)


@functools.cache
def _pair_gather_kernel():
    per_w = _PAIRS_P // (_NC * _NS)
    ntrip = per_w // _K
    f = 64
    shapes = [_NP["operation"], _NP["machine"], _NP["job"]]
    out = [jax.ShapeDtypeStruct((_PAIRS_P, f), jnp.float32) for _ in range(3)]
    scratch = [
        pltpu.VMEM_SHARED((shapes[1], f), jnp.float32),
        pltpu.VMEM_SHARED((shapes[2], f), jnp.float32),
        pltpu.VMEM((per_w,), jnp.int32),
        pltpu.VMEM((per_w,), jnp.int32),
        pltpu.VMEM((per_w,), jnp.int32),
        pltpu.VMEM((_K, f), jnp.float32),
        pltpu.VMEM((_K, f), jnp.float32),
        pltpu.VMEM((_K, f), jnp.float32),
        pltpu.SemaphoreType.DMA,
        pltpu.SemaphoreType.DMA,
        pltpu.SemaphoreType.DMA,
    ]
    mesh = plsc.VectorSubcoreMesh(
        core_axis_name="c", subcore_axis_name="s",
        num_cores=_NC, num_subcores=_NS)

    @functools.partial(
        pl.kernel, out_type=out, mesh=mesh, scratch_types=scratch,
        compiler_params=pltpu.CompilerParams(
            needs_layout_passes=False, use_tc_tiling_on_sc=False),
    )
    def k(t_op, t_ma, t_job, i0, i1, i2, g0, g1, g2,
          st_ma, st_job, b0, b1, b2, v0, v1, v2, s0, s1, s2):
        c = lax.axis_index("c")
        s = lax.axis_index("s")
        wid = s * _NC + c
        qm = shapes[1] // _NS
        qj = shapes[2] // _NS
        pltpu.sync_copy(t_ma.at[pl.ds(s * qm, qm)], st_ma.at[pl.ds(s * qm, qm)])
        pltpu.sync_copy(t_job.at[pl.ds(s * qj, qj)],
                        st_job.at[pl.ds(s * qj, qj)])
        plsc.subcore_barrier()

        base = wid * per_w
        pltpu.sync_copy(i0.at[pl.ds(base, per_w)], b0)
        pltpu.sync_copy(i1.at[pl.ds(base, per_w)], b1)
        pltpu.sync_copy(i2.at[pl.ds(base, per_w)], b2)

        def chunk(j, _):
            o = j * _K
            d0 = pltpu.async_copy(t_op.at[b0.at[pl.ds(o, _K)]], v0, s0)
            d1 = pltpu.async_copy(st_ma.at[b1.at[pl.ds(o, _K)]], v1, s1)
            d2 = pltpu.async_copy(st_job.at[b2.at[pl.ds(o, _K)]], v2, s2)
            d0.wait()
            d1.wait()
            d2.wait()
            pltpu.sync_copy(v0, g0.at[pl.ds(base + o, _K)])
            pltpu.sync_copy(v1, g1.at[pl.ds(base + o, _K)])
            pltpu.sync_copy(v2, g2.at[pl.ds(base + o, _K)])
            return 0

        lax.fori_loop(0, ntrip, chunk, 0)

    return k


def _row_mask(vals, base, n_valid):
    rid = base + lax.broadcasted_iota(jnp.int32, vals.shape, 0)
    return jnp.where(rid < n_valid, vals, 0.0)


@functools.cache
def _encoder_kernel(n_p, n, inch, bs):
    grid = n_p // bs

    def body(x_ref, w_ref, o_ref):
        i = pl.program_id(0)
        hx = jnp.dot(x_ref[...], w_ref[...],
                     preferred_element_type=jnp.float32)
        enc = jnp.concatenate([jnp.sin(hx), jnp.cos(hx)], axis=1)
        o_ref[...] = _row_mask(enc, i * bs, n)

    return pl.pallas_call(
        body,
        grid=(grid,),
        in_specs=[pl.BlockSpec((bs, inch), lambda i: (i, 0)),
                  pl.BlockSpec((inch, 16), lambda i: (0, 0))],
        out_specs=pl.BlockSpec((bs, 32), lambda i: (i, 0)),
        out_shape=jax.ShapeDtypeStruct((n_p, 32), jnp.float32),
    )


@functools.cache
def _conv3a_kernel(n_p, n, f_in, bs):
    grid = n_p // bs
    f = 64

    def body(z0_ref, z1_ref, z2_ref, w1_ref, b1_ref,
             h0_ref, h1_ref, h2_ref, s1_ref, s2_ref, ctr_ref):
        i = pl.program_id(0)
        ones = jnp.ones((1, bs), jnp.float32)
        hrefs = (h0_ref, h1_ref, h2_ref)

        @pl.when(i == 0)
        def _():
            s1_ref[...] = jnp.zeros_like(s1_ref)
            s2_ref[...] = jnp.zeros_like(s2_ref)

        for e, zr in enumerate((z0_ref, z1_ref, z2_ref)):
            he = jnp.dot(zr[...], w1_ref[e],
                         preferred_element_type=jnp.float32) + b1_ref[e, 0:1, :]
            he = _row_mask(he, i * bs, n)
            hrefs[e][...] = he

            @pl.when(i == 0)
            def _():
                cm = jnp.dot(ones, he, preferred_element_type=jnp.float32,
                             precision=lax.Precision.HIGHEST) * (1.0 / bs)
                ctr_ref[e] = jnp.broadcast_to(cm, (8, f))

            hc = _row_mask(he - ctr_ref[e, 0:1, :], i * bs, n)
            s1_ref[e, 0:1, :] += jnp.dot(
                ones, he, preferred_element_type=jnp.float32,
                precision=lax.Precision.HIGHEST)
            s2_ref[e, 0:1, :] += jnp.dot(
                ones, hc * hc, preferred_element_type=jnp.float32,
                precision=lax.Precision.HIGHEST)

    return pl.pallas_call(
        body,
        grid=(grid,),
        in_specs=[pl.BlockSpec((bs, f_in), lambda i: (i, 0))] * 3
        + [pl.BlockSpec((3, f_in, f), lambda i: (0, 0, 0)),
           pl.BlockSpec((3, 8, f), lambda i: (0, 0, 0))],
        out_specs=[pl.BlockSpec((bs, f), lambda i: (i, 0))] * 3
        + [pl.BlockSpec((3, 8, f), lambda i: (0, 0, 0)),
           pl.BlockSpec((3, 8, f), lambda i: (0, 0, 0)),
           pl.BlockSpec((3, 8, f), lambda i: (0, 0, 0))],
        out_shape=[jax.ShapeDtypeStruct((n_p, f), jnp.float32)] * 3
        + [jax.ShapeDtypeStruct((3, 8, f), jnp.float32)] * 3,
    )


@functools.cache
def _conv3b_kernel(n_p, n, bs, has_res):
    grid = n_p // bs
    f = 64

    def body(*refs):
        if has_res:
            h0_ref, h1_ref, h2_ref, res_ref, so_ref, w2_ref, b2_ref, o_ref = refs
        else:
            h0_ref, h1_ref, h2_ref, so_ref, w2_ref, b2_ref, o_ref = refs
            res_ref = None
        i = pl.program_id(0)
        acc = jnp.broadcast_to(b2_ref[0:1, :], (bs, f))
        if has_res:
            acc = acc + res_ref[...]
        for e, hr in enumerate((h0_ref, h1_ref, h2_ref)):
            hb = jnp.maximum(hr[...] * so_ref[e, 0:1, :] + so_ref[e, 1:2, :],
                             0.0)
            acc = acc + jnp.dot(hb, w2_ref[e],
                                preferred_element_type=jnp.float32)
        o_ref[...] = _row_mask(acc, i * bs, n)

    in_specs = [pl.BlockSpec((bs, f), lambda i: (i, 0))] * 3
    if has_res:
        in_specs.append(pl.BlockSpec((bs, f), lambda i: (i, 0)))
    in_specs += [
        pl.BlockSpec((3, 8, f), lambda i: (0, 0, 0)),
        pl.BlockSpec((3, f, f), lambda i: (0, 0, 0)),
        pl.BlockSpec((8, f), lambda i: (0, 0)),
    ]
    return pl.pallas_call(
        body,
        grid=(grid,),
        in_specs=in_specs,
        out_specs=pl.BlockSpec((bs, f), lambda i: (i, 0)),
        out_shape=jax.ShapeDtypeStruct((n_p, f), jnp.float32),
    )


@functools.cache
def _conv1_kernel(n_p, n, f_in, has_res):
    def body(*refs):
        if has_res:
            z_ref, res_ref, w1_ref, w2_ref, vec_ref, o_ref = refs
        else:
            z_ref, w1_ref, w2_ref, vec_ref, o_ref = refs
            res_ref = None
        b1 = vec_ref[0:1, :]
        g1 = vec_ref[1:2, :]
        be1 = vec_ref[2:3, :]
        b2 = vec_ref[3:4, :]
        z = z_ref[...]
        h1 = jnp.dot(z, w1_ref[...], preferred_element_type=jnp.float32) + b1
        npad = n_p - n
        mu = (h1.sum(0, keepdims=True) - npad * b1) * (1.0 / n)
        d = h1 - mu
        var = ((d * d).sum(0, keepdims=True)
               - npad * (b1 - mu) * (b1 - mu)) * (1.0 / n)
        hb = jnp.maximum(g1 * d * lax.rsqrt(var + _EPS) + be1, 0.0)
        out = jnp.dot(hb, w2_ref[...], preferred_element_type=jnp.float32) + b2
        if has_res:
            out = out + res_ref[...]
        o_ref[...] = _row_mask(out, 0, n)

    in_specs = [pl.BlockSpec((n_p, f_in), lambda: (0, 0))]
    if has_res:
        in_specs.append(pl.BlockSpec((n_p, 64), lambda: (0, 0)))
    in_specs += [
        pl.BlockSpec((f_in, 64), lambda: (0, 0)),
        pl.BlockSpec((64, 64), lambda: (0, 0)),
        pl.BlockSpec((8, 64), lambda: (0, 0)),
    ]
    return pl.pallas_call(
        body,
        grid=(),
        in_specs=in_specs,
        out_specs=pl.BlockSpec((n_p, 64), lambda: (0, 0)),
        out_shape=jax.ShapeDtypeStruct((n_p, 64), jnp.float32),
    )


@functools.cache
def _linear_kernel(n_p, f_in, f_out, bs):
    grid = n_p // bs

    def body(x_ref, w_ref, o_ref):
        o_ref[...] = jnp.dot(x_ref[...], w_ref[...],
                             preferred_element_type=jnp.float32)

    return pl.pallas_call(
        body,
        grid=(grid,),
        in_specs=[pl.BlockSpec((bs, f_in), lambda i: (i, 0)),
                  pl.BlockSpec((f_in, f_out), lambda i: (0, 0))],
        out_specs=pl.BlockSpec((bs, f_out), lambda i: (i, 0)),
        out_shape=jax.ShapeDtypeStruct((n_p, f_out), jnp.float32),
    )


@functools.cache
def _score_stats_kernel(bs):
    grid = _PAIRS_P // bs
    f = 64

    def body(g0_ref, g1_ref, g2_ref, b1_ref, h_ref, s1_ref, s2_ref,
             ctr_ref):
        i = pl.program_id(0)
        ones = jnp.ones((1, bs), jnp.float32)

        h = g0_ref[...] + g1_ref[...] + g2_ref[...] + b1_ref[0:1, :]
        h = _row_mask(h, i * bs, _PAIRS)
        h_ref[...] = h

        @pl.when(i == 0)
        def _():
            s1_ref[...] = jnp.zeros_like(s1_ref)
            s2_ref[...] = jnp.zeros_like(s2_ref)
            cm = jnp.dot(ones, h, preferred_element_type=jnp.float32,
                         precision=lax.Precision.HIGHEST) * (1.0 / bs)
            ctr_ref[...] = jnp.broadcast_to(cm, (8, f))

        hc = _row_mask(h - ctr_ref[0:1, :], i * bs, _PAIRS)
        s1_ref[0:1, :] += jnp.dot(ones, h, preferred_element_type=jnp.float32,
                                  precision=lax.Precision.HIGHEST)
        s2_ref[0:1, :] += jnp.dot(ones, hc * hc,
                                  preferred_element_type=jnp.float32,
                                  precision=lax.Precision.HIGHEST)

    return pl.pallas_call(
        body,
        grid=(grid,),
        in_specs=[pl.BlockSpec((bs, f), lambda i: (i, 0))] * 3
        + [pl.BlockSpec((8, f), lambda i: (0, 0))],
        out_specs=[pl.BlockSpec((bs, f), lambda i: (i, 0)),
                   pl.BlockSpec((8, f), lambda i: (0, 0)),
                   pl.BlockSpec((8, f), lambda i: (0, 0)),
                   pl.BlockSpec((8, f), lambda i: (0, 0))],
        out_shape=[jax.ShapeDtypeStruct((_PAIRS_P, f), jnp.float32),
                   jax.ShapeDtypeStruct((8, f), jnp.float32),
                   jax.ShapeDtypeStruct((8, f), jnp.float32),
                   jax.ShapeDtypeStruct((8, f), jnp.float32)],
    )


@functools.cache
def _score_pass2_kernel(bs):
    grid = _PAIRS_P // bs
    f, f2 = 64, 32

    def body(h_ref, sc_ref, of_ref, w2_ref, b2_ref, h2_ref, s1_ref, s2_ref,
             ctr_ref):
        i = pl.program_id(0)
        ones = jnp.ones((1, bs), jnp.float32)

        hb = jnp.maximum(h_ref[...] * sc_ref[0:1, :] + of_ref[0:1, :], 0.0)
        h2 = jnp.dot(hb, w2_ref[...],
                     preferred_element_type=jnp.float32) + b2_ref[0:1, :]
        h2 = _row_mask(h2, i * bs, _PAIRS)
        h2_ref[...] = h2

        @pl.when(i == 0)
        def _():
            s1_ref[...] = jnp.zeros_like(s1_ref)
            s2_ref[...] = jnp.zeros_like(s2_ref)
            cm = jnp.dot(ones, h2, preferred_element_type=jnp.float32,
                         precision=lax.Precision.HIGHEST) * (1.0 / bs)
            ctr_ref[...] = jnp.broadcast_to(cm, (8, f2))

        hc = _row_mask(h2 - ctr_ref[0:1, :], i * bs, _PAIRS)
        s1_ref[0:1, :] += jnp.dot(ones, h2, preferred_element_type=jnp.float32,
                                  precision=lax.Precision.HIGHEST)
        s2_ref[0:1, :] += jnp.dot(ones, hc * hc,
                                  preferred_element_type=jnp.float32,
                                  precision=lax.Precision.HIGHEST)

    return pl.pallas_call(
        body,
        grid=(grid,),
        in_specs=[pl.BlockSpec((bs, f), lambda i: (i, 0)),
                  pl.BlockSpec((8, f), lambda i: (0, 0)),
                  pl.BlockSpec((8, f), lambda i: (0, 0)),
                  pl.BlockSpec((f, f2), lambda i: (0, 0)),
                  pl.BlockSpec((8, f2), lambda i: (0, 0))],
        out_specs=[pl.BlockSpec((bs, f2), lambda i: (i, 0)),
                   pl.BlockSpec((8, f2), lambda i: (0, 0)),
                   pl.BlockSpec((8, f2), lambda i: (0, 0)),
                   pl.BlockSpec((8, f2), lambda i: (0, 0))],
        out_shape=[jax.ShapeDtypeStruct((_PAIRS_P, f2), jnp.float32),
                   jax.ShapeDtypeStruct((8, f2), jnp.float32),
                   jax.ShapeDtypeStruct((8, f2), jnp.float32),
                   jax.ShapeDtypeStruct((8, f2), jnp.float32)],
    )


@functools.cache
def _score_pass3_kernel(bs):
    grid = _PAIRS_P // bs
    f2 = 32

    def body(h2_ref, sc_ref, of_ref, w3_ref, b3_ref, o_ref):
        hb = jnp.maximum(h2_ref[...] * sc_ref[0:1, :] + of_ref[0:1, :], 0.0)
        o_ref[...] = jnp.dot(hb, w3_ref[...],
                             preferred_element_type=jnp.float32) + b3_ref[0:1, :]

    return pl.pallas_call(
        body,
        grid=(grid,),
        in_specs=[pl.BlockSpec((bs, f2), lambda i: (i, 0)),
                  pl.BlockSpec((8, f2), lambda i: (0, 0)),
                  pl.BlockSpec((8, f2), lambda i: (0, 0)),
                  pl.BlockSpec((f2, 1), lambda i: (0, 0)),
                  pl.BlockSpec((8, 1), lambda i: (0, 0))],
        out_specs=pl.BlockSpec((bs, 1), lambda i: (i, 0)),
        out_shape=jax.ShapeDtypeStruct((_PAIRS_P, 1), jnp.float32),
    )


def _vec8(*rows):
    f = rows[0].shape[-1]
    v = jnp.zeros((8, f), jnp.float32)
    for r, x in enumerate(rows):
        v = v.at[r].set(x.reshape(f))
    return v


def kernel(x_operation, x_machine, x_job, edge_index_operation_on_machine, edge_index_machine_rev_on_operation, edge_index_operation_belongs_job, edge_index_job_contains_operation, edge_index_operation_precedes_operation, valid_pairs, params):
    edge_dict = {
        _EDGE_TYPES[0]: edge_index_operation_on_machine,
        _EDGE_TYPES[1]: edge_index_machine_rev_on_operation,
        _EDGE_TYPES[2]: edge_index_operation_belongs_job,
        _EDGE_TYPES[3]: edge_index_job_contains_operation,
        _EDGE_TYPES[4]: edge_index_operation_precedes_operation,
    }
    raw = {"operation": x_operation, "machine": x_machine, "job": x_job}

    epad = {}
    for et in _EDGE_TYPES:
        ei = edge_dict[et]
        epad[et] = _pad_edges(ei, _e_pad(ei.shape[1]))

    x = {}
    for nt in _NODE_TYPES:
        xr = raw[nt]
        n, n_p = _N[nt], _NP[nt]
        xr_p = jnp.pad(xr, ((0, n_p - n), (0, 8 - xr.shape[1])))
        enc_p = jnp.pad(params["enc_" + nt], ((0, 8 - xr.shape[1]), (0, 0)))
        bs = {"operation": 1568, "machine": 1024, "job": 1024}[nt]
        x[nt] = _encoder_kernel(n_p, n, 8, bs)(xr_p, enc_p)

    op_ets = [_EDGE_TYPES[1], _EDGE_TYPES[3], _EDGE_TYPES[4]]
    prev = None
    for l in range(_NUM_LAYERS):
        f = 32 if l == 0 else 64
        z = {}
        for et in _EDGE_TYPES:
            src, _, dst = et
            row, col = epad[et]
            z[et] = _segsum(row, col, x[src], x[dst],
                            stage_src=(src == "machine"))

        x_new = {}
        zs = [z[et] for et in op_ets]
        pre = [params["conv%d_%s" % (l, "_".join(et))] for et in op_ets]
        w1s = jnp.stack([p["W1"] for p in pre])
        w2s = jnp.stack([p["W2"] for p in pre])
        b1s = jnp.stack([_vec8(p["b1"]) for p in pre])
        np_op, n_op = _NP["operation"], _N["operation"]
        h0, h1_, h2_, s1c, s2c, ctrc = _conv3a_kernel(np_op, n_op, f, 1568)(
            *zs, w1s, b1s)
        so = []
        for e, p in enumerate(pre):
            mu = s1c[e, 0] / n_op
            dd = mu - ctrc[e, 0]
            var = s2c[e, 0] / n_op - dd * dd
            scale = p["g1"] * lax.rsqrt(var + _EPS)
            so.append(jnp.stack([scale, p["be1"] - mu * scale]))
        sov = jnp.stack([jnp.pad(s_, ((0, 6), (0, 0))) for s_ in so])
        b2sum = _vec8(pre[0]["b2"] + pre[1]["b2"] + pre[2]["b2"])
        args = [h0, h1_, h2_] + ([prev["operation"]] if prev else []) + [
            sov, w2s, b2sum]
        x_new["operation"] = _conv3b_kernel(np_op, n_op, 1568,
                                            prev is not None)(*args)

        for nt, et in (("machine", _EDGE_TYPES[0]), ("job", _EDGE_TYPES[2])):
            p = params["conv%d_%s" % (l, "_".join(et))]
            vec = _vec8(p["b1"], p["g1"], p["be1"], p["b2"])
            args = [z[et]] + ([prev[nt]] if prev else []) + [p["W1"], p["W2"],
                                                             vec]
            x_new[nt] = _conv1_kernel(_NP[nt], _N[nt], f,
                                      prev is not None)(*args)
        prev = x_new
        x = x_new

    sp = params["score"]
    p_op = _linear_kernel(_NP["operation"], 64, 64, 1568)(
        x["operation"], sp["W1"][0:64])
    p_ma = _linear_kernel(_NP["machine"], 64, 64, 1024)(
        x["machine"], sp["W1"][64:128])
    p_job = _linear_kernel(_NP["job"], 64, 64, 1024)(
        x["job"], sp["W1"][128:192])

    idx = [jnp.pad(valid_pairs[:, j], (0, _PAIRS_P - _PAIRS)) for j in range(3)]
    g0, g1, g2 = _pair_gather_kernel()(p_op, p_ma, p_job, *idx)

    b1v = _vec8(sp["b1"])
    h1, s1, s2, c1 = _score_stats_kernel(1568)(g0, g1, g2, b1v)
    mu1 = s1[0] / _PAIRS
    d1 = mu1 - c1[0]
    var1 = s2[0] / _PAIRS - d1 * d1
    sc1 = sp["g1"] * lax.rsqrt(var1 + _EPS)
    of1 = sp["be1"] - mu1 * sc1

    h2, t1, t2, c2 = _score_pass2_kernel(1568)(
        h1, _vec8(sc1), _vec8(of1), sp["W2"], _vec8(sp["b2"]))
    mu2 = t1[0] / _PAIRS
    d2 = mu2 - c2[0]
    var2 = t2[0] / _PAIRS - d2 * d2
    sc2 = sp["g2"] * lax.rsqrt(var2 + _EPS)
    of2 = sp["be2"] - mu2 * sc2

    out = _score_pass3_kernel(1568)(
        h2, _vec8(sc2), _vec8(of2), sp["W3"],
        jnp.broadcast_to(sp["b3"].reshape(1, 1), (8, 1)))
    return out[:_PAIRS]
